# baseline (speedup 1.0000x reference)
import functools

import jax
import jax.numpy as jnp
from jax import lax
from jax.experimental import pallas as pl
from jax.experimental.pallas import tpu as pltpu
from jax.experimental.pallas import tpu_sc as plsc

GROUP = 16
NBUF = 2


@functools.lru_cache(maxsize=None)
def _make_gather(B: int, V: int, D: int):
  info = plsc.get_sparse_core_info()
  nw = info.num_cores * info.num_subcores
  assert B % (8 * nw) == 0
  b_per_w = B // nw
  n_groups = b_per_w // GROUP
  n_outer = n_groups // NBUF
  mesh = plsc.VectorSubcoreMesh(core_axis_name="c", subcore_axis_name="s")

  @functools.partial(
      pl.kernel,
      mesh=mesh,
      out_type=jax.ShapeDtypeStruct((D, B), jnp.float32),
      scratch_types=[
          pltpu.VMEM((b_per_w,), jnp.int32),
          pltpu.VMEM((D, GROUP * 128), jnp.float32),
          pltpu.VMEM((D, GROUP * 128), jnp.float32),
          pltpu.VMEM((D, b_per_w), jnp.float32),
          pltpu.SemaphoreType.DMA,
          pltpu.SemaphoreType.DMA,
      ],
      compiler_params=pltpu.CompilerParams(
          use_tc_tiling_on_sc=True,
          needs_layout_passes=False,
          disable_bounds_checks=True,
          disable_semaphore_checks=True,
      ),
  )
  def gather_kernel(table_t_hbm, idx_hbm, out_t_hbm, idx_v, blk0, blk1,
                    cols_v, sem0, sem1):
    wid = lax.axis_index("s") * info.num_cores + lax.axis_index("c")
    base = wid * b_per_w
    pltpu.sync_copy(idx_hbm.at[pl.ds(base, b_per_w)], idx_v)

    lane = lax.iota(jnp.int32, 16)
    blks = (blk0, blk1)
    sems = (sem0, sem1)

    def fire(g, b):
      idx_vec = idx_v[pl.ds(g * GROUP, GROUP)]
      for k in range(GROUP):
        i0 = pl.multiple_of((idx_vec[k] >> 7) << 7, 128)
        pltpu.async_copy(
            table_t_hbm.at[:, pl.ds(i0, 128)],
            blks[b].at[:, pl.ds(k * 128, 128)],
            sems[b],
        )

    def drain_and_pack(g, b):
      pltpu.make_async_copy(
          table_t_hbm.at[:, pl.ds(0, GROUP * 128)], blks[b], sems[b]
      ).wait()
      idx_vec = idx_v[pl.ds(g * GROUP, GROUP)]
      col = lane * 128 + (idx_vec & 127)
      for d in range(D):
        v = plsc.load_gather(blks[b], [jnp.full((16,), d, jnp.int32), col])
        plsc.store_scatter(cols_v, [jnp.full((16,), d, jnp.int32),
                                    lane + g * GROUP], v)

    for b in range(NBUF):
      fire(b, b)

    def outer(t):
      for b in range(NBUF):
        g = t * NBUF + b
        drain_and_pack(g, b)

        @pl.when(t < n_outer - 1)
        def _():
          fire(g + NBUF, b)

    pl.loop(0, n_outer)(outer)

    pltpu.sync_copy(cols_v, out_t_hbm.at[:, pl.ds(base, b_per_w)])

  return gather_kernel


def kernel(hw_ids, table):
  B, = hw_ids.shape
  V, D = table.shape
  out_t = _make_gather(B, V, D)(table.T, hw_ids.astype(jnp.int32))
  return out_t.T


# speedup vs baseline: 1.0582x; 1.0582x over previous
import functools

import jax
import jax.numpy as jnp
from jax import lax
from jax.experimental import pallas as pl
from jax.experimental.pallas import tpu as pltpu
from jax.experimental.pallas import tpu_sc as plsc

GROUP = 8
NBUF = 4


@functools.lru_cache(maxsize=None)
def _make_gather(B: int, V: int, D: int):
  info = plsc.get_sparse_core_info()
  nw = info.num_cores * info.num_subcores
  assert B % (8 * nw) == 0
  b_per_w = B // nw
  n_groups = b_per_w // GROUP
  n_outer = n_groups // NBUF
  per_outer = NBUF * GROUP
  mesh = plsc.VectorSubcoreMesh(core_axis_name="c", subcore_axis_name="s")

  @functools.partial(
      pl.kernel,
      mesh=mesh,
      out_type=jax.ShapeDtypeStruct((D, B), jnp.float32),
      scratch_types=[
          pltpu.VMEM((b_per_w + NBUF * GROUP,), jnp.int32),
          pltpu.VMEM((D, GROUP * 128), jnp.float32),
          pltpu.VMEM((D, GROUP * 128), jnp.float32),
          pltpu.VMEM((D, GROUP * 128), jnp.float32),
          pltpu.VMEM((D, GROUP * 128), jnp.float32),
          pltpu.VMEM((D, b_per_w), jnp.float32),
          pltpu.SemaphoreType.DMA,
          pltpu.SemaphoreType.DMA,
          pltpu.SemaphoreType.DMA,
          pltpu.SemaphoreType.DMA,
      ],
      compiler_params=pltpu.CompilerParams(
          use_tc_tiling_on_sc=True,
          needs_layout_passes=False,
          disable_bounds_checks=True,
          disable_semaphore_checks=True,
      ),
  )
  def gather_kernel(table_t_hbm, idx_hbm, out_t_hbm, idx_v, blk0, blk1,
                    blk2, blk3, cols_v, sem0, sem1, sem2, sem3):
    wid = lax.axis_index("s") * info.num_cores + lax.axis_index("c")
    base = wid * b_per_w
    pltpu.sync_copy(idx_hbm.at[pl.ds(base, b_per_w)],
                    idx_v.at[pl.ds(0, b_per_w)])

    lane = lax.iota(jnp.int32, 16)
    blks = (blk0, blk1, blk2, blk3)
    sems = (sem0, sem1, sem2, sem3)

    def windows(t):
      return [idx_v[pl.ds(t * per_outer + 16 * w, 16)]
              for w in range(per_outer // 16)]

    def fire(b, win, off):
      for k in range(GROUP):
        i0 = pl.multiple_of((win[off + k] >> 7) << 7, 128)
        pltpu.async_copy(
            table_t_hbm.at[:, pl.ds(i0, 128)],
            blks[b].at[:, pl.ds(k * 128, 128)],
            sems[b],
        )

    def drain(b):
      pltpu.make_async_copy(
          table_t_hbm.at[:, pl.ds(0, GROUP * 128)], blks[b], sems[b]
      ).wait()

    def pack(g, b, win, off):
      for k in range(GROUP):
        i = win[off + k]
        col = jnp.full((16,), i & 127, jnp.int32) + k * 128
        v = plsc.load_gather(blks[b], [lane, col])
        plsc.store_scatter(
            cols_v, [lane, jnp.full((16,), g * GROUP + k, jnp.int32)], v)

    w_first = windows(0)
    for b in range(NBUF):
      fire(b, w_first[(b * GROUP) // 16], (b * GROUP) % 16)

    def outer(t):
      cur = windows(t)
      nxt = windows(t + 1)
      for b in range(NBUF):
        g = t * NBUF + b
        w, off = (b * GROUP) // 16, (b * GROUP) % 16
        drain(b)
        pack(g, b, cur[w], off)

        @pl.when(t < n_outer - 1)
        def _():
          fire(b, nxt[w], off)

    pl.loop(0, n_outer)(outer)

    pltpu.sync_copy(cols_v, out_t_hbm.at[:, pl.ds(base, b_per_w)])

  return gather_kernel


def kernel(hw_ids, table):
  B, = hw_ids.shape
  V, D = table.shape
  out_t = _make_gather(B, V, D)(table.T, hw_ids.astype(jnp.int32))
  return out_t.T


# speedup vs baseline: 1.0608x; 1.0025x over previous
import functools

import jax
import jax.numpy as jnp
from jax import lax
from jax.experimental import pallas as pl
from jax.experimental.pallas import tpu as pltpu
from jax.experimental.pallas import tpu_sc as plsc

GROUP = 8
NBUF = 4


@functools.lru_cache(maxsize=None)
def _make_gather(B: int, V: int, D: int):
  info = plsc.get_sparse_core_info()
  nw = info.num_cores * info.num_subcores
  assert B % (8 * nw) == 0
  b_per_w = B // nw
  n_groups = b_per_w // GROUP
  n_outer = n_groups // NBUF
  per_outer = NBUF * GROUP
  mesh = plsc.VectorSubcoreMesh(core_axis_name="c", subcore_axis_name="s")

  @functools.partial(
      pl.kernel,
      mesh=mesh,
      out_type=jax.ShapeDtypeStruct((D, B), jnp.float32),
      scratch_types=[
          pltpu.VMEM((b_per_w + NBUF * GROUP,), jnp.int32),
          pltpu.VMEM((D, GROUP * 128), jnp.float32),
          pltpu.VMEM((D, GROUP * 128), jnp.float32),
          pltpu.VMEM((D, GROUP * 128), jnp.float32),
          pltpu.VMEM((D, GROUP * 128), jnp.float32),
          pltpu.VMEM((D, b_per_w), jnp.float32),
          pltpu.SemaphoreType.DMA,
          pltpu.SemaphoreType.DMA,
          pltpu.SemaphoreType.DMA,
          pltpu.SemaphoreType.DMA,
      ],
      compiler_params=pltpu.CompilerParams(
          use_tc_tiling_on_sc=True,
          needs_layout_passes=False,
          disable_bounds_checks=True,
          disable_semaphore_checks=True,
          skip_device_barrier=True,
      ),
  )
  def gather_kernel(table_t_hbm, idx_hbm, out_t_hbm, idx_v, blk0, blk1,
                    blk2, blk3, cols_v, sem0, sem1, sem2, sem3):
    wid = lax.axis_index("s") * info.num_cores + lax.axis_index("c")
    base = wid * b_per_w
    pltpu.sync_copy(idx_hbm.at[pl.ds(base, b_per_w)],
                    idx_v.at[pl.ds(0, b_per_w)])

    lane = lax.iota(jnp.int32, 16)
    blks = (blk0, blk1, blk2, blk3)
    sems = (sem0, sem1, sem2, sem3)

    def windows(t):
      return [idx_v[pl.ds(t * per_outer + 16 * w, 16)]
              for w in range(per_outer // 16)]

    def fire(b, win, off):
      for k in range(GROUP):
        i0 = pl.multiple_of((win[off + k] >> 7) << 7, 128)
        pltpu.async_copy(
            table_t_hbm.at[:, pl.ds(i0, 128)],
            blks[b].at[:, pl.ds(k * 128, 128)],
            sems[b],
        )

    def drain(b):
      pltpu.make_async_copy(
          table_t_hbm.at[:, pl.ds(0, GROUP * 128)], blks[b], sems[b]
      ).wait()

    def pack(g, b, win, off):
      for k in range(GROUP):
        i = win[off + k]
        col = jnp.full((16,), i & 127, jnp.int32) + k * 128
        v = plsc.load_gather(blks[b], [lane, col])
        plsc.store_scatter(
            cols_v, [lane, jnp.full((16,), g * GROUP + k, jnp.int32)], v)

    w_first = windows(0)
    for b in range(NBUF):
      fire(b, w_first[(b * GROUP) // 16], (b * GROUP) % 16)

    def outer(t):
      cur = windows(t)
      nxt = windows(t + 1)
      for b in range(NBUF):
        g = t * NBUF + b
        w, off = (b * GROUP) // 16, (b * GROUP) % 16
        drain(b)
        pack(g, b, cur[w], off)

        @pl.when(t < n_outer - 1)
        def _():
          fire(b, nxt[w], off)

    pl.loop(0, n_outer)(outer)

    pltpu.sync_copy(cols_v, out_t_hbm.at[:, pl.ds(base, b_per_w)])

  return gather_kernel


def kernel(hw_ids, table):
  B, = hw_ids.shape
  V, D = table.shape
  out_t = _make_gather(B, V, D)(table.T, hw_ids.astype(jnp.int32))
  return out_t.T
